# baseline (speedup 1.0000x reference)
import jax
import jax.numpy as jnp
from jax.experimental import pallas as pl
from jax.experimental.pallas import tpu as pltpu

_N = 64
_F = 16
_S = 4
_C = 32
_KN = 64
_UNITS = 256
_G = 16


def _gnn_fused_kernel(x_ref, amat_ref, obs_ref,
                      k1_ref, b1_ref, k2_ref, b2_ref, k3p_ref, b3r_ref,
                      wroot_ref, rootb_ref, attnk_ref, wd_ref, bd_ref,
                      out_ref):
    NN = _N * _N
    X = x_ref[...].reshape(_G * _N, _F)
    ev = obs_ref[:, 40:168, :].astype(jnp.bfloat16)
    ev = ev.reshape(_G, _N, 2, 128)
    E_jsi = jnp.transpose(ev, (0, 2, 3, 1)).reshape(_G * _N, _S, _N)
    amat_t = jnp.transpose(amat_ref[...], (0, 2, 1))
    maskmat = (amat_t > 0.5).astype(jnp.float32)

    H1 = jnp.maximum(
        jax.lax.dot_general(E_jsi, k1_ref[...], (((1,), (0,)), ((), ())),
                            preferred_element_type=jnp.float32)
        + b1_ref[...], 0.0).astype(jnp.bfloat16)
    H3 = jnp.maximum(
        jax.lax.dot_general(H1, k2_ref[...], (((2,), (0,)), ((), ())),
                            preferred_element_type=jnp.float32)
        + b2_ref[...], 0.0).astype(jnp.bfloat16)

    M3 = jnp.dot(X.astype(jnp.bfloat16), k3p_ref[...],
                 preferred_element_type=jnp.float32)
    M3 = M3.astype(jnp.bfloat16).reshape(_G * _N, _KN, _C)

    msumT = jax.lax.dot_general(
        M3, H3, (((1,), (2,)), ((0,), (0,))),
        preferred_element_type=jnp.float32)
    msumT = msumT * maskmat.reshape(_G * _N, 1, _N)
    aggT = jnp.sum(msumT.reshape(_G, _N, _C, _N), axis=1)
    agg = jnp.transpose(aggT, (0, 2, 1))

    Xb3 = jnp.dot(X, b3r_ref[...],
                  preferred_element_type=jnp.float32).reshape(_G, _N, _C)
    agg = agg + jax.lax.dot_general(
        maskmat.reshape(_G, _N, _N), Xb3, (((1,), (1,)), ((0,), (0,))),
        preferred_element_type=jnp.float32)

    XW = jnp.dot(X, wroot_ref[...],
                 preferred_element_type=jnp.float32).reshape(_G, _N, _C)
    Xc = jnp.maximum(agg + XW + rootb_ref[...], 0.0)

    lg = jnp.sum(Xc * attnk_ref[...], axis=2, keepdims=True)
    ex = jnp.exp(lg - jnp.max(lg, axis=1, keepdims=True))
    attn = ex / jnp.sum(ex, axis=1, keepdims=True)
    pooled = jnp.sum(attn * Xc, axis=1)

    out_ref[0] = jnp.tanh(
        jnp.dot(pooled, wd_ref[...], preferred_element_type=jnp.float32)
        + bd_ref[...])


def kernel(observations, K1, b1, K2, b2, K3, b3, W_root, root_bias,
           attn_k, Wd, bd):
    Bc = observations.shape[0]
    NF, NN = _N * _F, _N * _N

    Xr = observations[:, :NF].reshape(Bc, _N, _F)
    Araw = observations[:, NF:NF + NN].reshape(Bc, _N, _N)
    Obs3 = observations.reshape(Bc, 168, 128)

    K3p = K3.reshape(_KN, _F, _C).transpose(1, 0, 2).reshape(_F, _KN * _C)
    K1b = K1.astype(jnp.bfloat16)
    K2b = K2.astype(jnp.bfloat16)
    K3pb = K3p.astype(jnp.bfloat16)
    b3r = b3.reshape(_F, _C)

    b1r = b1.reshape(1, _KN)
    b2r = b2.reshape(1, _KN)
    rootbr = root_bias.reshape(1, _C)
    attnkr = attn_k.reshape(1, 1, _C)
    bdr = bd.reshape(1, _UNITS)

    def full(a):
        return pl.BlockSpec(a.shape, lambda b: (0,) * a.ndim)

    grid_spec = pl.GridSpec(
        grid=(Bc // _G,),
        in_specs=[
            pl.BlockSpec((_G, _N, _F), lambda b: (b, 0, 0)),
            pl.BlockSpec((_G, _N, _N), lambda b: (b, 0, 0)),
            pl.BlockSpec((_G, 168, 128), lambda b: (b, 0, 0)),
            full(K1b), full(b1r), full(K2b), full(b2r), full(K3pb), full(b3r),
            full(W_root), full(rootbr), full(attnkr), full(Wd), full(bdr),
        ],
        out_specs=pl.BlockSpec((1, _G, _UNITS), lambda b: (b, 0, 0)),
    )

    out = pl.pallas_call(
        _gnn_fused_kernel,
        grid_spec=grid_spec,
        out_shape=jax.ShapeDtypeStruct((Bc // _G, _G, _UNITS), jnp.float32),
        compiler_params=pltpu.CompilerParams(
            dimension_semantics=("parallel",)),
    )(Xr, Araw, Obs3, K1b, b1r, K2b, b2r, K3pb, b3r,
      W_root, rootbr, attnkr, Wd, bdr)
    return out.reshape(Bc, _UNITS)


# speedup vs baseline: 2.3996x; 2.3996x over previous
import jax
import jax.numpy as jnp
from jax.experimental import pallas as pl
from jax.experimental.pallas import tpu as pltpu

_N = 64
_F = 16
_S = 4
_C = 32
_KN = 64
_UNITS = 256
_G = 16


def _gnn_fused_kernel(x_ref, amat_ref, obs_ref,
                      k1_ref, b1_ref, k2_ref, b2_ref, k3p_ref, b3r_ref,
                      wroot_ref, rootb_ref, attnk_ref, wd_ref, bd_ref,
                      out_ref):
    NN = _N * _N
    X = x_ref[...].reshape(_G * _N, _F)
    et = jnp.transpose(obs_ref[...].astype(jnp.bfloat16), (0, 2, 1))
    E_jsi = et.reshape(_G * _N, _S, _N)
    amat_t = jnp.transpose(amat_ref[...], (0, 2, 1))
    maskmat = (amat_t > 0.5).astype(jnp.float32)

    H1 = jnp.maximum(
        jax.lax.dot_general(E_jsi, k1_ref[...], (((1,), (0,)), ((), ())),
                            preferred_element_type=jnp.float32)
        + b1_ref[...], 0.0).astype(jnp.bfloat16)
    H3 = jnp.maximum(
        jax.lax.dot_general(H1, k2_ref[...], (((2,), (0,)), ((), ())),
                            preferred_element_type=jnp.float32)
        + b2_ref[...], 0.0).astype(jnp.bfloat16)

    M3 = jnp.dot(X.astype(jnp.bfloat16), k3p_ref[...],
                 preferred_element_type=jnp.float32)
    M3 = M3.astype(jnp.bfloat16).reshape(_G * _N, _KN, _C)

    msumT = jax.lax.dot_general(
        M3, H3, (((1,), (2,)), ((0,), (0,))),
        preferred_element_type=jnp.float32)
    msumT = msumT * maskmat.reshape(_G * _N, 1, _N)
    aggT = jnp.sum(msumT.reshape(_G, _N, _C, _N), axis=1)
    agg = jnp.transpose(aggT, (0, 2, 1))

    Xb3 = jnp.dot(X, b3r_ref[...],
                  preferred_element_type=jnp.float32).reshape(_G, _N, _C)
    agg = agg + jax.lax.dot_general(
        maskmat.reshape(_G, _N, _N), Xb3, (((1,), (1,)), ((0,), (0,))),
        preferred_element_type=jnp.float32)

    XW = jnp.dot(X, wroot_ref[...],
                 preferred_element_type=jnp.float32).reshape(_G, _N, _C)
    Xc = jnp.maximum(agg + XW + rootb_ref[...], 0.0)

    lg = jnp.sum(Xc * attnk_ref[...], axis=2, keepdims=True)
    ex = jnp.exp(lg - jnp.max(lg, axis=1, keepdims=True))
    attn = ex / jnp.sum(ex, axis=1, keepdims=True)
    pooled = jnp.sum(attn * Xc, axis=1)

    out_ref[0] = jnp.tanh(
        jnp.dot(pooled, wd_ref[...], preferred_element_type=jnp.float32)
        + bd_ref[...])


def kernel(observations, K1, b1, K2, b2, K3, b3, W_root, root_bias,
           attn_k, Wd, bd):
    Bc = observations.shape[0]
    NF, NN = _N * _F, _N * _N

    Xr = observations[:, :NF].reshape(Bc, _N, _F)
    Araw = observations[:, NF:NF + NN].reshape(Bc, _N, _N)
    Obs3 = observations[:, NF + NN:].reshape(Bc, _N, _N * _S)

    K3p = K3.reshape(_KN, _F, _C).transpose(1, 0, 2).reshape(_F, _KN * _C)
    K1b = K1.astype(jnp.bfloat16)
    K2b = K2.astype(jnp.bfloat16)
    K3pb = K3p.astype(jnp.bfloat16)
    b3r = b3.reshape(_F, _C)

    b1r = b1.reshape(1, _KN)
    b2r = b2.reshape(1, _KN)
    rootbr = root_bias.reshape(1, _C)
    attnkr = attn_k.reshape(1, 1, _C)
    bdr = bd.reshape(1, _UNITS)

    def full(a):
        return pl.BlockSpec(a.shape, lambda b: (0,) * a.ndim)

    grid_spec = pl.GridSpec(
        grid=(Bc // _G,),
        in_specs=[
            pl.BlockSpec((_G, _N, _F), lambda b: (b, 0, 0)),
            pl.BlockSpec((_G, _N, _N), lambda b: (b, 0, 0)),
            pl.BlockSpec((_G, _N, _N * _S), lambda b: (b, 0, 0)),
            full(K1b), full(b1r), full(K2b), full(b2r), full(K3pb), full(b3r),
            full(W_root), full(rootbr), full(attnkr), full(Wd), full(bdr),
        ],
        out_specs=pl.BlockSpec((1, _G, _UNITS), lambda b: (b, 0, 0)),
    )

    out = pl.pallas_call(
        _gnn_fused_kernel,
        grid_spec=grid_spec,
        out_shape=jax.ShapeDtypeStruct((Bc // _G, _G, _UNITS), jnp.float32),
        compiler_params=pltpu.CompilerParams(
            dimension_semantics=("parallel",)),
    )(Xr, Araw, Obs3, K1b, b1r, K2b, b2r, K3pb, b3r,
      W_root, rootbr, attnkr, Wd, bdr)
    return out.reshape(Bc, _UNITS)


# speedup vs baseline: 2.4286x; 1.0121x over previous
import jax
import jax.numpy as jnp
from jax.experimental import pallas as pl
from jax.experimental.pallas import tpu as pltpu

_N = 64
_F = 16
_S = 4
_C = 32
_KN = 64
_UNITS = 256
_G = 16


def _gnn_fused_kernel(x_ref, amat_ref, obs_ref,
                      k1_ref, b1_ref, k2_ref, b2_ref, k3p_ref,
                      wcat_ref, rootb_ref, attnk_ref, wd_ref, bd_ref,
                      out_ref):
    NN = _N * _N
    X = x_ref[...].reshape(_G * _N, _F)
    et = jnp.transpose(obs_ref[...], (0, 2, 1))
    E_jsi = et.reshape(_G * _N, _S, _N)
    amat_t = jnp.transpose(amat_ref[...], (0, 2, 1))
    maskmat = (amat_t > 0.5).astype(jnp.float32)

    H1 = jnp.maximum(
        jax.lax.dot_general(E_jsi, k1_ref[...], (((1,), (0,)), ((), ())),
                            preferred_element_type=jnp.float32)
        + b1_ref[...], 0.0).astype(jnp.bfloat16)
    H3 = jnp.maximum(
        jax.lax.dot_general(H1, k2_ref[...], (((2,), (0,)), ((), ())),
                            preferred_element_type=jnp.float32)
        + b2_ref[...], 0.0).astype(jnp.bfloat16)

    M3 = jnp.dot(X.astype(jnp.bfloat16), k3p_ref[...],
                 preferred_element_type=jnp.float32)
    M3 = M3.astype(jnp.bfloat16).reshape(_G * _N, _KN, _C)

    msumT = jax.lax.dot_general(
        M3, H3, (((1,), (2,)), ((0,), (0,))),
        preferred_element_type=jnp.float32)
    msumT = msumT * maskmat.reshape(_G * _N, 1, _N)
    aggT = jnp.sum(msumT.reshape(_G, _N, _C, _N), axis=1)
    agg = jnp.transpose(aggT, (0, 2, 1))

    XWb = jnp.dot(X, wcat_ref[...],
                  preferred_element_type=jnp.float32)
    XW = XWb[:, :_C].reshape(_G, _N, _C)
    Xb3 = XWb[:, _C:].reshape(_G, _N, _C)
    agg = agg + jax.lax.dot_general(
        maskmat.reshape(_G, _N, _N), Xb3, (((1,), (1,)), ((0,), (0,))),
        preferred_element_type=jnp.float32)
    Xc = jnp.maximum(agg + XW + rootb_ref[...], 0.0)

    lg = jnp.sum(Xc * attnk_ref[...], axis=2, keepdims=True)
    ex = jnp.exp(lg - jnp.max(lg, axis=1, keepdims=True))
    attn = ex / jnp.sum(ex, axis=1, keepdims=True)
    pooled = jnp.sum(attn * Xc, axis=1)

    out_ref[0] = jnp.tanh(
        jnp.dot(pooled, wd_ref[...], preferred_element_type=jnp.float32)
        + bd_ref[...])


def kernel(observations, K1, b1, K2, b2, K3, b3, W_root, root_bias,
           attn_k, Wd, bd):
    Bc = observations.shape[0]
    NF, NN = _N * _F, _N * _N

    Xr = observations[:, :NF].reshape(Bc, _N, _F)
    Araw = observations[:, NF:NF + NN].reshape(Bc, _N, _N)
    Obs3 = observations[:, NF + NN:].reshape(
        Bc, _N, _N * _S).astype(jnp.bfloat16)

    K3p = K3.reshape(_KN, _F, _C).transpose(1, 0, 2).reshape(_F, _KN * _C)
    K1b = K1.astype(jnp.bfloat16)
    K2b = K2.astype(jnp.bfloat16)
    K3pb = K3p.astype(jnp.bfloat16)
    Wcat = jnp.concatenate([W_root, b3.reshape(_F, _C)], axis=1)

    b1r = b1.reshape(1, _KN)
    b2r = b2.reshape(1, _KN)
    rootbr = root_bias.reshape(1, _C)
    attnkr = attn_k.reshape(1, 1, _C)
    bdr = bd.reshape(1, _UNITS)

    def full(a):
        return pl.BlockSpec(a.shape, lambda b: (0,) * a.ndim)

    grid_spec = pl.GridSpec(
        grid=(Bc // _G,),
        in_specs=[
            pl.BlockSpec((_G, _N, _F), lambda b: (b, 0, 0)),
            pl.BlockSpec((_G, _N, _N), lambda b: (b, 0, 0)),
            pl.BlockSpec((_G, _N, _N * _S), lambda b: (b, 0, 0)),
            full(K1b), full(b1r), full(K2b), full(b2r), full(K3pb),
            full(Wcat), full(rootbr), full(attnkr), full(Wd), full(bdr),
        ],
        out_specs=pl.BlockSpec((1, _G, _UNITS), lambda b: (b, 0, 0)),
    )

    out = pl.pallas_call(
        _gnn_fused_kernel,
        grid_spec=grid_spec,
        out_shape=jax.ShapeDtypeStruct((Bc // _G, _G, _UNITS), jnp.float32),
        compiler_params=pltpu.CompilerParams(
            dimension_semantics=("parallel",)),
    )(Xr, Araw, Obs3, K1b, b1r, K2b, b2r, K3pb,
      Wcat, rootbr, attnkr, Wd, bdr)
    return out.reshape(Bc, _UNITS)
